# baseline (speedup 1.0000x reference)
import functools

import jax
import jax.numpy as jnp
from jax import lax
from jax.experimental import pallas as pl
from jax.experimental.pallas import tpu as pltpu
from jax.experimental.pallas import tpu_sc as plsc

_B, _T, _D, _F, _MEL = 16, 512, 256, 256, 4096
_EPS = 1e-5
_NC, _NS = 2, 16
_NW = _NC * _NS
_HALF = _MEL // 2
_CHUNK = 256
_ZROW = _B * _T
_SENT = _T + 1


def _dur_body(x_ref, w1_ref, b1_ref, g1_ref, bb1_ref, w2_ref, b2_ref,
              g2_ref, bb2_ref, lw_ref, lb_ref, out_ref):
    x = x_ref[0]
    zrow = jnp.zeros((1, _D), jnp.float32)

    def conv_relu_ln(h, w_ref, b_ref, g_ref, beta_ref):
        hm = jnp.concatenate([zrow, h[:-1]], axis=0)
        hp = jnp.concatenate([h[1:], zrow], axis=0)
        w = w_ref[...]
        o = (jnp.dot(hm, w[0], preferred_element_type=jnp.float32)
             + jnp.dot(h, w[1], preferred_element_type=jnp.float32)
             + jnp.dot(hp, w[2], preferred_element_type=jnp.float32))
        o = jnp.maximum(o + b_ref[...], 0.0)
        mu = jnp.mean(o, axis=-1, keepdims=True)
        var = jnp.mean(jnp.square(o - mu), axis=-1, keepdims=True)
        return (o - mu) * jax.lax.rsqrt(var + _EPS) * g_ref[...] + beta_ref[...]

    h = conv_relu_ln(x, w1_ref, b1_ref, g1_ref, bb1_ref)
    h = conv_relu_ln(h, w2_ref, b2_ref, g2_ref, bb2_ref)
    dur = jnp.sum(h * lw_ref[...], axis=-1, keepdims=True) + lb_ref[...]
    out_ref[0] = jnp.maximum(dur, 0.0)


def _duration_predictor(x, w1, b1, g1, bb1, w2, b2, g2, bb2, lw, lb):
    full = lambda shape: pl.BlockSpec(shape, lambda i: (0,) * len(shape))
    return pl.pallas_call(
        _dur_body,
        grid=(_B,),
        in_specs=[
            pl.BlockSpec((1, _T, _D), lambda i: (i, 0, 0)),
            full((3, _D, _F)), full((1, _F)), full((1, _F)), full((1, _F)),
            full((3, _F, _F)), full((1, _F)), full((1, _F)), full((1, _F)),
            full((1, _F)), full((1, 1)),
        ],
        out_specs=pl.BlockSpec((1, _T, 1), lambda i: (i, 0, 0)),
        out_shape=jax.ShapeDtypeStruct((_B, _T, 1), jnp.float32),
    )(x, w1, b1, g1, bb1, w2, b2, g2, bb2, lw, lb)


def _expand_body(xpad_hbm, tgt_hbm, out_hbm, tgt_v, idx_v, rows_v, sem):
    w = lax.axis_index("s") * _NC + lax.axis_index("c")
    b = w // 2
    half = w % 2
    iota = lax.iota(jnp.int32, 16)

    pltpu.sync_copy(tgt_hbm.at[b], tgt_v)

    zero16 = jnp.zeros((16,), jnp.int32)

    def zbody(j, c):
        idx_v[pl.ds(j * 16, 16)] = zero16
        return c

    lax.fori_loop(0, (_MEL + 16) // 16, zbody, 0, unroll=4)

    def sbody(j, carry):
        tv = tgt_v[pl.ds(j * 16, 16)]
        cs = plsc.cumsum(tv) + carry
        cstart = cs - tv
        val = j * 16 + iota + 1
        plsc.store_scatter(idx_v, [cstart], val, mask=tv > 0)
        return jnp.max(cs)

    total = lax.fori_loop(0, _T // 16, sbody, jnp.int32(0))

    plsc.store_scatter(idx_v, [jnp.full((16,), total, jnp.int32)],
                       jnp.full((16,), _SENT, jnp.int32), mask=iota == 0)

    def fbody(j, carry):
        v = jnp.maximum(idx_v[pl.ds(j * 16, 16)], carry)
        m = plsc.cummax(v)
        g = jnp.where((m >= 1) & (m <= _T), b * _T + m - 1, _ZROW)
        idx_v[pl.ds(j * 16, 16)] = g
        return jnp.max(m)

    lax.fori_loop(0, _MEL // 16, fbody, jnp.int32(0), unroll=2)

    row0 = half * _HALF
    out_base = b * _MEL + row0
    for i in range(_HALF // _CHUNK):
        idx_chunk = idx_v.at[pl.ds(row0 + i * _CHUNK, _CHUNK)]
        pltpu.async_copy(xpad_hbm.at[idx_chunk], rows_v, sem).wait()
        pltpu.sync_copy(rows_v,
                        out_hbm.at[pl.ds(out_base + i * _CHUNK, _CHUNK)])


@functools.cache
def _expand():
    return pl.kernel(
        _expand_body,
        out_type=jax.ShapeDtypeStruct((_B * _MEL, _D), jnp.float32),
        mesh=plsc.VectorSubcoreMesh(core_axis_name="c", subcore_axis_name="s",
                                    num_cores=_NC, num_subcores=_NS),
        compiler_params=pltpu.CompilerParams(needs_layout_passes=False),
        scratch_types=[
            pltpu.VMEM((_T,), jnp.int32),
            pltpu.VMEM((_MEL + 16,), jnp.int32),
            pltpu.VMEM((_CHUNK, _D), jnp.float32),
            pltpu.SemaphoreType.DMA,
        ],
    )


def kernel(x, conv1_w, conv1_b, ln1_g, ln1_b, conv2_w, conv2_b, ln2_g,
           ln2_b, lin_w, lin_b, target, mel_max_length):
    w1 = jnp.transpose(conv1_w, (2, 1, 0))
    w2 = jnp.transpose(conv2_w, (2, 1, 0))
    row = lambda v: v.reshape(1, -1)
    dur = _duration_predictor(
        x, w1, row(conv1_b), row(ln1_g), row(ln1_b),
        w2, row(conv2_b), row(ln2_g), row(ln2_b),
        row(lin_w), lin_b.reshape(1, 1))[:, :, 0]

    xpad = jnp.concatenate(
        [x.reshape(_B * _T, _D), jnp.zeros((16, _D), x.dtype)], axis=0)
    out = _expand()(xpad, target).reshape(_B, _MEL, _D)
    return (out, dur)


# speedup vs baseline: 14.7773x; 14.7773x over previous
import functools

import jax
import jax.numpy as jnp
from jax import lax
from jax.experimental import pallas as pl
from jax.experimental.pallas import tpu as pltpu
from jax.experimental.pallas import tpu_sc as plsc

_B, _T, _D, _F, _MEL = 16, 512, 256, 256, 4096
_EPS = 1e-5
_NC, _NS = 2, 16
_NW = _NC * _NS
_HALF = _MEL // 2
_CHUNK = 128
_ZROW = _B * _T
_NZPAD = 128
_SENT = _T + 1


def _dur_body(x_ref, w1_ref, b1_ref, g1_ref, bb1_ref, w2_ref, b2_ref,
              g2_ref, bb2_ref, lw_ref, lb_ref, out_ref):
    x = x_ref[0]
    zrow = jnp.zeros((1, _D), jnp.float32)

    def conv_relu_ln(h, w_ref, b_ref, g_ref, beta_ref):
        hm = jnp.concatenate([zrow, h[:-1]], axis=0)
        hp = jnp.concatenate([h[1:], zrow], axis=0)
        w = w_ref[...]
        o = (jnp.dot(hm, w[0], preferred_element_type=jnp.float32)
             + jnp.dot(h, w[1], preferred_element_type=jnp.float32)
             + jnp.dot(hp, w[2], preferred_element_type=jnp.float32))
        o = jnp.maximum(o + b_ref[...], 0.0)
        mu = jnp.mean(o, axis=-1, keepdims=True)
        var = jnp.mean(jnp.square(o - mu), axis=-1, keepdims=True)
        return (o - mu) * jax.lax.rsqrt(var + _EPS) * g_ref[...] + beta_ref[...]

    h = conv_relu_ln(x, w1_ref, b1_ref, g1_ref, bb1_ref)
    h = conv_relu_ln(h, w2_ref, b2_ref, g2_ref, bb2_ref)
    dur = jnp.sum(h * lw_ref[...], axis=-1, keepdims=True) + lb_ref[...]
    out_ref[0] = jnp.maximum(dur, 0.0)


def _duration_predictor(x, w1, b1, g1, bb1, w2, b2, g2, bb2, lw, lb):
    full = lambda shape: pl.BlockSpec(shape, lambda i: (0,) * len(shape))
    return pl.pallas_call(
        _dur_body,
        grid=(_B,),
        in_specs=[
            pl.BlockSpec((1, _T, _D), lambda i: (i, 0, 0)),
            full((3, _D, _F)), full((1, _F)), full((1, _F)), full((1, _F)),
            full((3, _F, _F)), full((1, _F)), full((1, _F)), full((1, _F)),
            full((1, _F)), full((1, 1)),
        ],
        out_specs=pl.BlockSpec((1, _T, 1), lambda i: (i, 0, 0)),
        out_shape=jax.ShapeDtypeStruct((_B, _T, 1), jnp.float32),
    )(x, w1, b1, g1, bb1, w2, b2, g2, bb2, lw, lb)


def _expand_body(xpad_hbm, tgt_hbm, out_hbm, tgt_v, idx_v, rows_v, zero_v,
                 sem):
    w = lax.axis_index("s") * _NC + lax.axis_index("c")
    b = w // 2
    half = w % 2
    iota = lax.iota(jnp.int32, 16)

    pltpu.sync_copy(tgt_hbm.at[b], tgt_v)
    pltpu.sync_copy(xpad_hbm.at[pl.ds(_ZROW, _CHUNK)], zero_v)

    zero16 = jnp.zeros((16,), jnp.int32)

    def zbody(j, c):
        idx_v[pl.ds(j * 16, 16)] = zero16
        return c

    lax.fori_loop(0, (_MEL + 16) // 16, zbody, 0, unroll=4)

    def sbody(j, carry):
        tv = tgt_v[pl.ds(j * 16, 16)]
        cs = plsc.cumsum(tv) + carry
        cstart = cs - tv
        val = j * 16 + iota + 1
        plsc.store_scatter(idx_v, [cstart], val, mask=tv > 0)
        return jnp.max(cs)

    total = lax.fori_loop(0, _T // 16, sbody, jnp.int32(0))

    plsc.store_scatter(idx_v, [jnp.full((16,), total, jnp.int32)],
                       jnp.full((16,), _SENT, jnp.int32), mask=iota == 0)

    def fbody(j, carry):
        v = jnp.maximum(idx_v[pl.ds(j * 16, 16)], carry)
        m = plsc.cummax(v)
        zspread = _ZROW + (j % (_NZPAD // 16)) * 16 + iota
        g = jnp.where((m >= 1) & (m <= _T), b * _T + m - 1, zspread)
        idx_v[pl.ds(j * 16, 16)] = g
        return jnp.max(m)

    lax.fori_loop(0, _MEL // 16, fbody, jnp.int32(0), unroll=2)

    row0 = half * _HALF
    out_base = b * _MEL + row0
    for i in range(_HALF // _CHUNK):
        start = row0 + i * _CHUNK
        out_slice = out_hbm.at[pl.ds(out_base + i * _CHUNK, _CHUNK)]

        @pl.when(total > start)
        def _():
            idx_chunk = idx_v.at[pl.ds(start, _CHUNK)]
            pltpu.async_copy(xpad_hbm.at[idx_chunk], rows_v, sem).wait()
            pltpu.sync_copy(rows_v, out_slice)

        @pl.when(total <= start)
        def _():
            pltpu.sync_copy(zero_v, out_slice)


@functools.cache
def _expand():
    return pl.kernel(
        _expand_body,
        out_type=jax.ShapeDtypeStruct((_B * _MEL, _D), jnp.float32),
        mesh=plsc.VectorSubcoreMesh(core_axis_name="c", subcore_axis_name="s",
                                    num_cores=_NC, num_subcores=_NS),
        compiler_params=pltpu.CompilerParams(needs_layout_passes=False),
        scratch_types=[
            pltpu.VMEM((_T,), jnp.int32),
            pltpu.VMEM((_MEL + 16,), jnp.int32),
            pltpu.VMEM((_CHUNK, _D), jnp.float32),
            pltpu.VMEM((_CHUNK, _D), jnp.float32),
            pltpu.SemaphoreType.DMA,
        ],
    )


def kernel(x, conv1_w, conv1_b, ln1_g, ln1_b, conv2_w, conv2_b, ln2_g,
           ln2_b, lin_w, lin_b, target, mel_max_length):
    w1 = jnp.transpose(conv1_w, (2, 1, 0))
    w2 = jnp.transpose(conv2_w, (2, 1, 0))
    row = lambda v: v.reshape(1, -1)
    dur = _duration_predictor(
        x, w1, row(conv1_b), row(ln1_g), row(ln1_b),
        w2, row(conv2_b), row(ln2_g), row(ln2_b),
        row(lin_w), lin_b.reshape(1, 1))[:, :, 0]

    xpad = jnp.concatenate(
        [x.reshape(_B * _T, _D), jnp.zeros((_NZPAD + 16, _D), x.dtype)],
        axis=0)
    out = _expand()(xpad, target).reshape(_B, _MEL, _D)
    return (out, dur)


# speedup vs baseline: 17.4760x; 1.1826x over previous
import functools

import jax
import jax.numpy as jnp
from jax import lax
from jax.experimental import pallas as pl
from jax.experimental.pallas import tpu as pltpu
from jax.experimental.pallas import tpu_sc as plsc

_B, _T, _D, _F, _MEL = 16, 512, 256, 256, 4096
_EPS = 1e-5
_NC, _NS = 2, 16
_NW = _NC * _NS
_HALF = _MEL // 2
_CHUNK = 128
_ZROW = _B * _T
_NZPAD = 128
_SENT = _T + 1


def _dur_body(x_ref, w1_ref, b1_ref, g1_ref, bb1_ref, w2_ref, b2_ref,
              g2_ref, bb2_ref, lw_ref, lb_ref, out_ref):
    x = x_ref[0]
    zrow = jnp.zeros((1, _D), jnp.float32)

    def conv_relu_ln(h, w_ref, b_ref, g_ref, beta_ref):
        hm = jnp.concatenate([zrow, h[:-1]], axis=0)
        hp = jnp.concatenate([h[1:], zrow], axis=0)
        w = w_ref[...]
        o = (jnp.dot(hm, w[0], preferred_element_type=jnp.float32)
             + jnp.dot(h, w[1], preferred_element_type=jnp.float32)
             + jnp.dot(hp, w[2], preferred_element_type=jnp.float32))
        o = jnp.maximum(o + b_ref[...], 0.0)
        mu = jnp.mean(o, axis=-1, keepdims=True)
        var = jnp.mean(jnp.square(o - mu), axis=-1, keepdims=True)
        return (o - mu) * jax.lax.rsqrt(var + _EPS) * g_ref[...] + beta_ref[...]

    h = conv_relu_ln(x, w1_ref, b1_ref, g1_ref, bb1_ref)
    h = conv_relu_ln(h, w2_ref, b2_ref, g2_ref, bb2_ref)
    dur = jnp.sum(h * lw_ref[...], axis=-1, keepdims=True) + lb_ref[...]
    out_ref[0] = jnp.maximum(dur, 0.0)


def _duration_predictor(x, w1, b1, g1, bb1, w2, b2, g2, bb2, lw, lb):
    full = lambda shape: pl.BlockSpec(shape, lambda i: (0,) * len(shape))
    return pl.pallas_call(
        _dur_body,
        grid=(_B,),
        in_specs=[
            pl.BlockSpec((1, _T, _D), lambda i: (i, 0, 0)),
            full((3, _D, _F)), full((1, _F)), full((1, _F)), full((1, _F)),
            full((3, _F, _F)), full((1, _F)), full((1, _F)), full((1, _F)),
            full((1, _F)), full((1, 1)),
        ],
        out_specs=pl.BlockSpec((1, _T, 1), lambda i: (i, 0, 0)),
        out_shape=jax.ShapeDtypeStruct((_B, _T, 1), jnp.float32),
    )(x, w1, b1, g1, bb1, w2, b2, g2, bb2, lw, lb)


def _expand_body(xpad_hbm, tgt_hbm, out_hbm, tgt_v, idx_v, rows_v, zero_v,
                 sem):
    w = lax.axis_index("s") * _NC + lax.axis_index("c")
    b = w // 2
    half = w % 2
    iota = lax.iota(jnp.int32, 16)

    pltpu.sync_copy(tgt_hbm.at[b], tgt_v)
    pltpu.sync_copy(xpad_hbm.at[pl.ds(_ZROW, _CHUNK)], zero_v)

    zero16 = jnp.zeros((16,), jnp.int32)

    def zbody(j, c):
        idx_v[pl.ds(j * 16, 16)] = zero16
        return c

    lax.fori_loop(0, (_MEL + 16) // 16, zbody, 0, unroll=4)

    def sbody(j, carry):
        tv = tgt_v[pl.ds(j * 16, 16)]
        cs = plsc.cumsum(tv) + carry
        cstart = cs - tv
        val = j * 16 + iota + 1
        plsc.store_scatter(idx_v, [cstart], val, mask=tv > 0)
        return jnp.max(cs)

    total = lax.fori_loop(0, _T // 16, sbody, jnp.int32(0))

    plsc.store_scatter(idx_v, [jnp.full((16,), total, jnp.int32)],
                       jnp.full((16,), _SENT, jnp.int32), mask=iota == 0)

    def fbody(j, carry):
        v = jnp.maximum(idx_v[pl.ds(j * 16, 16)], carry)
        m = plsc.cummax(v)
        zspread = _ZROW + (j % (_NZPAD // 16)) * 16 + iota
        g = jnp.where((m >= 1) & (m <= _T), b * _T + m - 1, zspread)
        idx_v[pl.ds(j * 16, 16)] = g
        return jnp.max(m)

    lax.fori_loop(0, _MEL // 16, fbody, jnp.int32(0), unroll=2)

    for i in range(_HALF // _CHUNK):
        start = (half + 2 * i) * _CHUNK
        out_slice = out_hbm.at[pl.ds(b * _MEL + start, _CHUNK)]

        @pl.when(total > start)
        def _():
            idx_chunk = idx_v.at[pl.ds(start, _CHUNK)]
            pltpu.async_copy(xpad_hbm.at[idx_chunk], rows_v, sem).wait()
            pltpu.sync_copy(rows_v, out_slice)

        @pl.when(total <= start)
        def _():
            pltpu.sync_copy(zero_v, out_slice)


@functools.cache
def _expand():
    return pl.kernel(
        _expand_body,
        out_type=jax.ShapeDtypeStruct((_B * _MEL, _D), jnp.float32),
        mesh=plsc.VectorSubcoreMesh(core_axis_name="c", subcore_axis_name="s",
                                    num_cores=_NC, num_subcores=_NS),
        compiler_params=pltpu.CompilerParams(needs_layout_passes=False),
        scratch_types=[
            pltpu.VMEM((_T,), jnp.int32),
            pltpu.VMEM((_MEL + 16,), jnp.int32),
            pltpu.VMEM((_CHUNK, _D), jnp.float32),
            pltpu.VMEM((_CHUNK, _D), jnp.float32),
            pltpu.SemaphoreType.DMA,
        ],
    )


def kernel(x, conv1_w, conv1_b, ln1_g, ln1_b, conv2_w, conv2_b, ln2_g,
           ln2_b, lin_w, lin_b, target, mel_max_length):
    w1 = jnp.transpose(conv1_w, (2, 1, 0))
    w2 = jnp.transpose(conv2_w, (2, 1, 0))
    row = lambda v: v.reshape(1, -1)
    dur = _duration_predictor(
        x, w1, row(conv1_b), row(ln1_g), row(ln1_b),
        w2, row(conv2_b), row(ln2_g), row(ln2_b),
        row(lin_w), lin_b.reshape(1, 1))[:, :, 0]

    xpad = jnp.concatenate(
        [x.reshape(_B * _T, _D), jnp.zeros((_NZPAD + 16, _D), x.dtype)],
        axis=0)
    out = _expand()(xpad, target).reshape(_B, _MEL, _D)
    return (out, dur)


# speedup vs baseline: 18.9155x; 1.0824x over previous
import functools

import jax
import jax.numpy as jnp
from jax import lax
from jax.experimental import pallas as pl
from jax.experimental.pallas import tpu as pltpu
from jax.experimental.pallas import tpu_sc as plsc

_B, _T, _D, _F, _MEL = 16, 512, 256, 256, 4096
_EPS = 1e-5
_NC, _NS = 2, 16
_NW = _NC * _NS
_HALF = _MEL // 2
_CHUNK = 128
_ZROW = _B * _T
_NZPAD = 128
_SENT = _T + 1


def _dur_body(x_ref, w1_ref, b1_ref, g1_ref, bb1_ref, w2_ref, b2_ref,
              g2_ref, bb2_ref, lw_ref, lb_ref, out_ref):
    x = x_ref[0]
    zrow = jnp.zeros((1, _D), jnp.float32)

    def conv_relu_ln(h, w_ref, b_ref, g_ref, beta_ref):
        hm = jnp.concatenate([zrow, h[:-1]], axis=0)
        hp = jnp.concatenate([h[1:], zrow], axis=0)
        w = w_ref[...]
        o = (jnp.dot(hm, w[0], preferred_element_type=jnp.float32)
             + jnp.dot(h, w[1], preferred_element_type=jnp.float32)
             + jnp.dot(hp, w[2], preferred_element_type=jnp.float32))
        o = jnp.maximum(o + b_ref[...], 0.0)
        mu = jnp.mean(o, axis=-1, keepdims=True)
        var = jnp.mean(jnp.square(o - mu), axis=-1, keepdims=True)
        return (o - mu) * jax.lax.rsqrt(var + _EPS) * g_ref[...] + beta_ref[...]

    h = conv_relu_ln(x, w1_ref, b1_ref, g1_ref, bb1_ref)
    h = conv_relu_ln(h, w2_ref, b2_ref, g2_ref, bb2_ref)
    dur = jnp.sum(h * lw_ref[...], axis=-1, keepdims=True) + lb_ref[...]
    out_ref[0] = jnp.maximum(dur, 0.0)


def _duration_predictor(x, w1, b1, g1, bb1, w2, b2, g2, bb2, lw, lb):
    full = lambda shape: pl.BlockSpec(shape, lambda i: (0,) * len(shape))
    return pl.pallas_call(
        _dur_body,
        grid=(_B,),
        in_specs=[
            pl.BlockSpec((1, _T, _D), lambda i: (i, 0, 0)),
            full((3, _D, _F)), full((1, _F)), full((1, _F)), full((1, _F)),
            full((3, _F, _F)), full((1, _F)), full((1, _F)), full((1, _F)),
            full((1, _F)), full((1, 1)),
        ],
        out_specs=pl.BlockSpec((1, _T, 1), lambda i: (i, 0, 0)),
        out_shape=jax.ShapeDtypeStruct((_B, _T, 1), jnp.float32),
    )(x, w1, b1, g1, bb1, w2, b2, g2, bb2, lw, lb)


def _expand_body(xpad_hbm, tgt_hbm, out_hbm, tgt_v, idx_v, rows_v, rows2_v,
                 zero_v, gsem0, gsem1, wsem0, wsem1, zsem):
    w = lax.axis_index("s") * _NC + lax.axis_index("c")
    b = w // 2
    half = w % 2
    iota = lax.iota(jnp.int32, 16)

    pltpu.sync_copy(tgt_hbm.at[b], tgt_v)
    pltpu.sync_copy(xpad_hbm.at[pl.ds(_ZROW, _CHUNK)], zero_v)

    zero16 = jnp.zeros((16,), jnp.int32)

    def zbody(j, c):
        idx_v[pl.ds(j * 16, 16)] = zero16
        return c

    lax.fori_loop(0, (_MEL + 16) // 16, zbody, 0, unroll=4)

    def sbody(j, carry):
        tv = tgt_v[pl.ds(j * 16, 16)]
        cs = plsc.cumsum(tv) + carry
        cstart = cs - tv
        val = j * 16 + iota + 1
        plsc.store_scatter(idx_v, [cstart], val, mask=tv > 0)
        return jnp.max(cs)

    total = lax.fori_loop(0, _T // 16, sbody, jnp.int32(0))

    plsc.store_scatter(idx_v, [jnp.full((16,), total, jnp.int32)],
                       jnp.full((16,), _SENT, jnp.int32), mask=iota == 0)

    def fbody(j, carry):
        v = jnp.maximum(idx_v[pl.ds(j * 16, 16)], carry)
        m = plsc.cummax(v)
        zspread = _ZROW + (j % (_NZPAD // 16)) * 16 + iota
        g = jnp.where((m >= 1) & (m <= _T), b * _T + m - 1, zspread)
        idx_v[pl.ds(j * 16, 16)] = g
        return jnp.max(m)

    lax.fori_loop(0, _MEL // 16, fbody, jnp.int32(0), unroll=2)

    n = _HALF // _CHUNK
    bufs = (rows_v, rows2_v)
    gsems = (gsem0, gsem1)
    wsems = (wsem0, wsem1)
    reals, gds, wds, zds = [], [], [], []
    for i in range(n):
        p = i % 2
        start = (half + 2 * i) * _CHUNK
        out_slice = out_hbm.at[pl.ds(b * _MEL + start, _CHUNK)]
        idx_chunk = idx_v.at[pl.ds(start, _CHUNK)]
        reals.append(total > start)
        gds.append(pltpu.make_async_copy(xpad_hbm.at[idx_chunk], bufs[p],
                                         gsems[p]))
        wds.append(pltpu.make_async_copy(bufs[p], out_slice, wsems[p]))
        zds.append(pltpu.make_async_copy(zero_v, out_slice, zsem))

    for i in range(n):
        @pl.when(reals[i])
        def _(i=i):
            if i >= 2:
                wds[i - 2].wait()
            gds[i].start()

        if i >= 1:
            @pl.when(reals[i - 1])
            def _(i=i):
                gds[i - 1].wait()
                wds[i - 1].start()

        @pl.when(jnp.logical_not(reals[i]))
        def _(i=i):
            zds[i].start()

    @pl.when(reals[n - 1])
    def _():
        gds[n - 1].wait()
        wds[n - 1].start()

    for i in range(n):
        last = (reals[i] & jnp.logical_not(reals[i + 2])
                if i + 2 < n else reals[i])

        @pl.when(last)
        def _(i=i):
            wds[i].wait()

        @pl.when(jnp.logical_not(reals[i]))
        def _(i=i):
            zds[i].wait()


@functools.cache
def _expand():
    return pl.kernel(
        _expand_body,
        out_type=jax.ShapeDtypeStruct((_B * _MEL, _D), jnp.float32),
        mesh=plsc.VectorSubcoreMesh(core_axis_name="c", subcore_axis_name="s",
                                    num_cores=_NC, num_subcores=_NS),
        compiler_params=pltpu.CompilerParams(needs_layout_passes=False),
        scratch_types=[
            pltpu.VMEM((_T,), jnp.int32),
            pltpu.VMEM((_MEL + 16,), jnp.int32),
            pltpu.VMEM((_CHUNK, _D), jnp.float32),
            pltpu.VMEM((_CHUNK, _D), jnp.float32),
            pltpu.VMEM((_CHUNK, _D), jnp.float32),
            pltpu.SemaphoreType.DMA,
            pltpu.SemaphoreType.DMA,
            pltpu.SemaphoreType.DMA,
            pltpu.SemaphoreType.DMA,
            pltpu.SemaphoreType.DMA,
        ],
    )


def kernel(x, conv1_w, conv1_b, ln1_g, ln1_b, conv2_w, conv2_b, ln2_g,
           ln2_b, lin_w, lin_b, target, mel_max_length):
    w1 = jnp.transpose(conv1_w, (2, 1, 0))
    w2 = jnp.transpose(conv2_w, (2, 1, 0))
    row = lambda v: v.reshape(1, -1)
    dur = _duration_predictor(
        x, w1, row(conv1_b), row(ln1_g), row(ln1_b),
        w2, row(conv2_b), row(ln2_g), row(ln2_b),
        row(lin_w), lin_b.reshape(1, 1))[:, :, 0]

    xpad = jnp.concatenate(
        [x.reshape(_B * _T, _D), jnp.zeros((_NZPAD + 16, _D), x.dtype)],
        axis=0)
    out = _expand()(xpad, target).reshape(_B, _MEL, _D)
    return (out, dur)


# speedup vs baseline: 19.9497x; 1.0547x over previous
import functools

import jax
import jax.numpy as jnp
from jax import lax
from jax.experimental import pallas as pl
from jax.experimental.pallas import tpu as pltpu
from jax.experimental.pallas import tpu_sc as plsc

_B, _T, _D, _F, _MEL = 16, 512, 256, 256, 4096
_EPS = 1e-5
_NC, _NS = 2, 16
_NW = _NC * _NS
_HALF = _MEL // 2
_CHUNK = 128
_ZROW = _B * _T
_NZPAD = 128
_SENT = _T + 1


def _dur_body(x_ref, w1_ref, b1_ref, g1_ref, bb1_ref, w2_ref, b2_ref,
              g2_ref, bb2_ref, lw_ref, lb_ref, out_ref):
    x = x_ref[0]
    zrow = jnp.zeros((1, _D), jnp.float32)

    def conv_relu_ln(h, w_ref, b_ref, g_ref, beta_ref):
        hm = jnp.concatenate([zrow, h[:-1]], axis=0)
        hp = jnp.concatenate([h[1:], zrow], axis=0)
        w = w_ref[...]
        o = (jnp.dot(hm, w[0], preferred_element_type=jnp.float32)
             + jnp.dot(h, w[1], preferred_element_type=jnp.float32)
             + jnp.dot(hp, w[2], preferred_element_type=jnp.float32))
        o = jnp.maximum(o + b_ref[...], 0.0)
        mu = jnp.mean(o, axis=-1, keepdims=True)
        var = jnp.mean(jnp.square(o - mu), axis=-1, keepdims=True)
        return (o - mu) * jax.lax.rsqrt(var + _EPS) * g_ref[...] + beta_ref[...]

    h = conv_relu_ln(x, w1_ref, b1_ref, g1_ref, bb1_ref)
    h = conv_relu_ln(h, w2_ref, b2_ref, g2_ref, bb2_ref)
    dur = jnp.sum(h * lw_ref[...], axis=-1, keepdims=True) + lb_ref[...]
    out_ref[0] = jnp.maximum(dur, 0.0)


def _duration_predictor(x, w1, b1, g1, bb1, w2, b2, g2, bb2, lw, lb):
    full = lambda shape: pl.BlockSpec(shape, lambda i: (0,) * len(shape))
    return pl.pallas_call(
        _dur_body,
        grid=(_B,),
        in_specs=[
            pl.BlockSpec((1, _T, _D), lambda i: (i, 0, 0)),
            full((3, _D, _F)), full((1, _F)), full((1, _F)), full((1, _F)),
            full((3, _F, _F)), full((1, _F)), full((1, _F)), full((1, _F)),
            full((1, _F)), full((1, 1)),
        ],
        out_specs=pl.BlockSpec((1, _T, 1), lambda i: (i, 0, 0)),
        out_shape=jax.ShapeDtypeStruct((_B, _T, 1), jnp.float32),
    )(x, w1, b1, g1, bb1, w2, b2, g2, bb2, lw, lb)


def _expand_body(x_hbm, tgt_hbm, zeros_hbm, out_hbm, tgt_v, idx_v, rows_v,
                 rows2_v, zero_v, gsem0, gsem1, wsem0, wsem1, zsem, ssem):
    w = lax.axis_index("s") * _NC + lax.axis_index("c")
    b = w // 2
    half = w % 2
    iota = lax.iota(jnp.int32, 16)

    zstage = pltpu.make_async_copy(zeros_hbm, zero_v, ssem)
    zstage.start()
    pltpu.sync_copy(tgt_hbm.at[b], tgt_v)

    zero16 = jnp.zeros((16,), jnp.int32)

    def zbody(j, c):
        idx_v[pl.ds(j * 16, 16)] = zero16
        return c

    lax.fori_loop(0, (_MEL + 16) // 16, zbody, 0, unroll=4)

    def sbody(j, carry):
        tv = tgt_v[pl.ds(j * 16, 16)]
        cs = plsc.cumsum(tv) + carry
        cstart = cs - tv
        val = j * 16 + iota + 1
        plsc.store_scatter(idx_v, [cstart], val, mask=tv > 0)
        return jnp.max(cs)

    total = lax.fori_loop(0, _T // 16, sbody, jnp.int32(0))

    plsc.store_scatter(idx_v, [jnp.full((16,), total, jnp.int32)],
                       jnp.full((16,), _SENT, jnp.int32), mask=iota == 0)

    n = _HALF // _CHUNK
    bufs = (rows_v, rows2_v)
    gsems = (gsem0, gsem1)
    wsems = (wsem0, wsem1)
    reals, gds, wds, zds = [], [], [], []
    for i in range(n):
        p = i % 2
        start = (half + 2 * i) * _CHUNK
        out_slice = out_hbm.at[pl.ds(b * _MEL + start, _CHUNK)]
        idx_chunk = idx_v.at[pl.ds(start, _CHUNK)]
        reals.append(total > start)
        gds.append(pltpu.make_async_copy(x_hbm.at[idx_chunk], bufs[p],
                                         gsems[p]))
        wds.append(pltpu.make_async_copy(bufs[p], out_slice, wsems[p]))
        zds.append(pltpu.make_async_copy(zero_v, out_slice, zsem))

    zstage.wait()
    for i in range(n):
        @pl.when(jnp.logical_not(reals[i]))
        def _(i=i):
            zds[i].start()

    def fbody(j, carry):
        v = jnp.maximum(idx_v[pl.ds(j * 16, 16)], carry)
        m = plsc.cummax(v)
        spread = b * _T + (j % 32) * 16 + iota
        g = jnp.where((m >= 1) & (m <= _T), b * _T + m - 1, spread)
        idx_v[pl.ds(j * 16, 16)] = g
        return jnp.max(m)

    fcarry = jnp.int32(0)
    group = (_MEL // 16) // n
    for i in range(n):
        fcarry = lax.fori_loop(i * group, (i + 1) * group, fbody, fcarry,
                               unroll=2)

    zrow16 = jnp.zeros((16,), jnp.float32)

    def _finish(i):
        gds[i].wait()
        start = (half + 2 * i) * _CHUNK
        buf = bufs[i % 2]

        @pl.when(total < start + _CHUNK)
        def _():
            def tz(r, c):
                for k in range(_D // 16):
                    buf[r, pl.ds(k * 16, 16)] = zrow16
                return c

            lax.fori_loop(total - start, _CHUNK, tz, 0)

        wds[i].start()

    for i in range(n):
        @pl.when(reals[i])
        def _(i=i):
            if i >= 2:
                wds[i - 2].wait()
            gds[i].start()

        if i >= 1:
            @pl.when(reals[i - 1])
            def _(i=i):
                _finish(i - 1)

    @pl.when(reals[n - 1])
    def _():
        _finish(n - 1)

    for i in range(n):
        last = (reals[i] & jnp.logical_not(reals[i + 2])
                if i + 2 < n else reals[i])

        @pl.when(last)
        def _(i=i):
            wds[i].wait()

        @pl.when(jnp.logical_not(reals[i]))
        def _(i=i):
            zds[i].wait()


@functools.cache
def _expand():
    return pl.kernel(
        _expand_body,
        out_type=jax.ShapeDtypeStruct((_B * _MEL, _D), jnp.float32),
        mesh=plsc.VectorSubcoreMesh(core_axis_name="c", subcore_axis_name="s",
                                    num_cores=_NC, num_subcores=_NS),
        compiler_params=pltpu.CompilerParams(needs_layout_passes=False),
        scratch_types=[
            pltpu.VMEM((_T,), jnp.int32),
            pltpu.VMEM((_MEL + 16,), jnp.int32),
            pltpu.VMEM((_CHUNK, _D), jnp.float32),
            pltpu.VMEM((_CHUNK, _D), jnp.float32),
            pltpu.VMEM((_CHUNK, _D), jnp.float32),
            pltpu.SemaphoreType.DMA,
            pltpu.SemaphoreType.DMA,
            pltpu.SemaphoreType.DMA,
            pltpu.SemaphoreType.DMA,
            pltpu.SemaphoreType.DMA,
            pltpu.SemaphoreType.DMA,
        ],
    )


def kernel(x, conv1_w, conv1_b, ln1_g, ln1_b, conv2_w, conv2_b, ln2_g,
           ln2_b, lin_w, lin_b, target, mel_max_length):
    w1 = jnp.transpose(conv1_w, (2, 1, 0))
    w2 = jnp.transpose(conv2_w, (2, 1, 0))
    row = lambda v: v.reshape(1, -1)
    dur = _duration_predictor(
        x, w1, row(conv1_b), row(ln1_g), row(ln1_b),
        w2, row(conv2_b), row(ln2_g), row(ln2_b),
        row(lin_w), lin_b.reshape(1, 1))[:, :, 0]

    zeros = jnp.zeros((_CHUNK, _D), jnp.float32)
    out = _expand()(x.reshape(_B * _T, _D), target, zeros)
    out = out.reshape(_B, _MEL, _D)
    return (out, dur)
